# baseline (speedup 1.0000x reference)
import jax
import jax.numpy as jnp
from jax.experimental import pallas as pl
from jax.experimental.pallas import tpu as pltpu

N = 16384
M = 4096
KNN = 20
MB = 512
RCH = 2048
GCH = 256
INT_ITERS = 9


def _block_kernel(aux_ref, xt_ref, out_ref, d2_ref, gm_ref):
    b0 = xt_ref[0:1, :]
    b1 = xt_ref[1:2, :]
    sub = RCH // GCH

    def _dist(c, _):
        a0 = aux_ref[pl.ds(c * RCH, RCH), 0:1]
        a1 = aux_ref[pl.ds(c * RCH, RCH), 1:2]
        d2c = (a0 - b0) ** 2 + (a1 - b1) ** 2
        d2_ref[pl.ds(c * RCH, RCH), :] = d2c
        gm_ref[pl.ds(c * sub, sub), :] = jnp.min(
            d2c.reshape(sub, GCH, MB), axis=1)
        return 0

    jax.lax.fori_loop(0, N // RCH, _dist, 0)

    gm = gm_ref[:, :]
    tl = jnp.maximum(jnp.min(gm, axis=0, keepdims=True), 1e-12)

    def _ext(i, carry):
        cmw, _ = carry
        mn = jnp.min(cmw, axis=0, keepdims=True)
        return jnp.where(cmw == mn, jnp.inf, cmw), mn

    _, th = jax.lax.fori_loop(0, KNN, _ext,
                              (gm, jnp.zeros((1, MB), jnp.float32)))
    th = th * 1.0001

    def _count(t):
        def _cnt(c, acc):
            blk = d2_ref[pl.ds(c * RCH, RCH), :]
            return acc + jnp.sum((blk < t).astype(jnp.float32), axis=0,
                                 keepdims=True)
        return jax.lax.fori_loop(0, N // RCH, _cnt,
                                 jnp.zeros((1, MB), jnp.float32))

    tgt = KNN - 0.5
    cl = jnp.zeros((1, MB), jnp.float32)
    ch = _count(th)

    def _interp(i, carry):
        tl, cl, th, ch, last = carry
        w = th - tl
        t = tl + (tgt - cl) * w / jnp.maximum(ch - cl, 1e-30)
        t = jnp.clip(t, tl + 0.01 * w, th - 0.01 * w)
        c = _count(t)
        up = c >= KNN
        tl2 = jnp.where(up, tl, t)
        cl2 = jnp.where(up, cl, c)
        th2 = jnp.where(up, t, th)
        ch2 = jnp.where(up, c, ch)
        cl2 = jnp.where(up & (last > 0), tgt + (cl2 - tgt) * 0.5, cl2)
        ch2 = jnp.where((~up) & (last < 0), tgt + (ch2 - tgt) * 0.5, ch2)
        return tl2, cl2, th2, ch2, jnp.where(up, 1.0, -1.0)

    last = jnp.zeros((1, MB), jnp.float32)
    tl, cl, th, ch, last = jax.lax.fori_loop(
        0, INT_ITERS, _interp, (tl, cl, th, ch, last))
    w = th - tl
    bw2 = tl + (tgt - cl) * w / jnp.maximum(ch - cl, 1e-30)
    bw2 = jnp.clip(bw2, tl, th)
    neg_half_inv_bw2 = -0.5 / bw2

    def _acc(c, carry):
        s, wy = carry
        wgt = jnp.exp(d2_ref[pl.ds(c * RCH, RCH), :] * neg_half_inv_bw2)
        y = aux_ref[pl.ds(c * RCH, RCH), 2:3]
        return (s + jnp.sum(wgt, axis=0, keepdims=True),
                wy + jnp.sum(wgt * y, axis=0, keepdims=True))

    zero = jnp.zeros((1, MB), jnp.float32)
    s, wy = jax.lax.fori_loop(0, N // RCH, _acc, (zero, zero))
    out_ref[:, :] = wy / (s + 1e-7)


@jax.jit
def kernel(inputs, outputs, x):
    aux = jnp.concatenate([inputs, outputs[:, None]], axis=1)
    xt = x.T
    out = pl.pallas_call(
        _block_kernel,
        grid=(M // MB,),
        in_specs=[
            pl.BlockSpec((N, 3), lambda i: (0, 0)),
            pl.BlockSpec((2, MB), lambda i: (0, i)),
        ],
        out_specs=pl.BlockSpec((1, MB), lambda i: (0, i)),
        out_shape=jax.ShapeDtypeStruct((1, M), jnp.float32),
        scratch_shapes=[pltpu.VMEM((N, MB), jnp.float32),
                        pltpu.VMEM((N // GCH, MB), jnp.float32)],
        compiler_params=pltpu.CompilerParams(
            dimension_semantics=("parallel",)),
    )(aux, xt)
    return out.reshape(M)
